# baseline (speedup 1.0000x reference)
import functools
import math

import jax
import jax.numpy as jnp
from jax import lax
from jax.experimental import pallas as pl
from jax.experimental.pallas import tpu as pltpu
from jax.experimental.pallas import tpu_sc as plsc

_F32 = jnp.float32
_NC, _NS = 2, 16
_CK = 128


def _ceil_to(a, m):
    return (a + m - 1) // m * m


def _mesh():
    return plsc.VectorSubcoreMesh(
        core_axis_name="c", subcore_axis_name="s",
        num_cores=_NC, num_subcores=_NS)


_ECK = 128
_K = 1
_CKE = 112
_ZC = 64


def _deg_call(dstp, rpad, ept32):
    ngr = ept32 // (_K * _ECK)
    rows_pt = rpad // _NS
    nch_r = rows_pt // _ECK

    @functools.partial(
        pl.kernel,
        out_type=(jax.ShapeDtypeStruct((rpad, 16), _F32),
                  jax.ShapeDtypeStruct((rpad, 16), _F32)),
        mesh=_mesh(),
        scratch_types=(
            [pltpu.VMEM((_ECK,), jnp.int32)] * _K
            + [pltpu.VMEM((_ECK, 16), _F32)] * 2
            + [pltpu.VMEM_SHARED((rpad, 16), _F32),
               pltpu.SemaphoreType.DMA]
        ),
    )
    def degk(dstr, out0r, out1r, db0, zbuf, obuf, acc, ssem):
        dbufs = (db0,)
        c = lax.axis_index("c")
        s = lax.axis_index("s")
        tb = s * rows_pt
        eb = (c * _NS + s) * ept32

        def fill(i, _):
            zbuf[i, pl.ds(0, 16)] = jnp.zeros((16,), _F32)
            obuf[i, pl.ds(0, 16)] = jnp.ones((16,), _F32)
            return _
        lax.fori_loop(0, _ECK, fill, None)

        def zcp(k, _):
            pltpu.sync_copy(zbuf, acc.at[pl.ds(tb + k * _ECK, _ECK)])
            return _
        lax.fori_loop(0, nch_r, zcp, None)
        plsc.subcore_barrier()

        def step(g, _):
            for k in range(_K):
                pltpu.sync_copy(
                    dstr.at[pl.ds(eb + (g * _K + k) * _ECK, _ECK)], dbufs[k])
            descs = [pltpu.async_copy(obuf, acc.at[dbufs[k]], ssem, add=True)
                     for k in range(_K)]
            for d in descs:
                d.wait()
            return _
        lax.fori_loop(0, ngr, step, None)
        plsc.subcore_barrier()

        def wout(outr):
            def w1(k, _):
                r0 = tb + k * _ECK
                pltpu.sync_copy(acc.at[pl.ds(r0, _ECK)], zbuf)
                pltpu.sync_copy(zbuf, outr.at[pl.ds(r0, _ECK)])
                return _
            lax.fori_loop(0, nch_r, w1, None)
        pl.when(c == 0)(lambda: wout(out0r))
        pl.when(c == 1)(lambda: wout(out1r))

    return degk(dstp)


def _spmm_call(hs0, hs1, src2, dst2, zer, rpad, ept):
    ngr = ept // _CKE
    rows_pt = rpad // _NS
    nzw = rows_pt // _ZC

    @functools.partial(
        pl.kernel,
        out_type=(jax.ShapeDtypeStruct((rpad, 128), _F32),
                  jax.ShapeDtypeStruct((rpad, 128), _F32)),
        mesh=_mesh(),
        scratch_types=(
            [pltpu.VMEM((_CKE, 128), _F32)] * 3
            + [pltpu.VMEM((_CKE,), jnp.int32)] * 9
            + [pltpu.VMEM_SHARED((rpad, 128), _F32)]
            + [pltpu.SemaphoreType.DMA] * 10
        ),
    )
    def spmm(hs0r, hs1r, srcr, dstr, zerr, out0r, out1r,
             b0, b1, b2, sb0, sb1, sb2, sb3, db0, db1, db2, db3, tbuf,
             acc, gsm0, gsm1, gsm2, ssm0, ssm1, ssm2,
             ism0, ism1, ism2, ism3):
        dbk = (b0, b1, b2)
        gsems = (gsm0, gsm1, gsm2)
        ssems = (ssm0, ssm1, ssm2)
        sidx = (sb0, sb1, sb2, sb3)
        didx = (db0, db1, db2, db3)
        isems = (ism0, ism1, ism2, ism3)
        c = lax.axis_index("c")
        s = lax.axis_index("s")
        tb = s * rows_pt
        eb = s * ept
        trash = rpad - 1

        def tfill(i, _):
            tbuf[pl.ds(i * 16, 16)] = jnp.full((16,), trash, jnp.int32)
            return _
        lax.fori_loop(0, _CKE // 16, tfill, None)

        pltpu.sync_copy(zerr, acc.at[pl.ds(tb, rows_pt)])
        plsc.subcore_barrier()

        def idx_load(i4, g):
            pltpu.async_copy(srcr.at[pl.ds(eb + g * _CKE, _CKE)],
                             sidx[i4], isems[i4])
            pltpu.async_copy(dstr.at[pl.ds(eb + g * _CKE, _CKE)],
                             didx[i4], isems[i4])

        def idx_drain(i4, g):
            pltpu.make_async_copy(srcr.at[pl.ds(eb + g * _CKE, _CKE)],
                                  sidx[i4], isems[i4]).wait()
            pltpu.make_async_copy(dstr.at[pl.ds(eb + g * _CKE, _CKE)],
                                  didx[i4], isems[i4]).wait()

        def run(hsr, outr):
            idx_load(0, 0)
            idx_load(1, 1)
            idx_load(2, 2)
            idx_drain(0, 0)
            pltpu.async_copy(hsr.at[sidx[0]], dbk[0], gsems[0])
            idx_drain(1, 1)
            pltpu.async_copy(hsr.at[sidx[1]], dbk[1], gsems[1])
            pltpu.async_copy(dbk[2], acc.at[tbuf], ssems[2], add=True)

            def jbody(j, _):
                for p in range(12):
                    g = j * 12 + p
                    d3, i4 = p % 3, p % 4
                    d3n2, i4n2, i4n3 = (p + 2) % 3, (p + 2) % 4, (p + 3) % 4
                    gn2 = lax.rem(g + 2, ngr)
                    gn3 = lax.rem(g + 3, ngr)
                    pltpu.make_async_copy(
                        hsr.at[sidx[i4]], dbk[d3], gsems[d3]).wait()
                    pltpu.async_copy(
                        dbk[d3], acc.at[didx[i4]], ssems[d3], add=True)
                    pltpu.make_async_copy(
                        dbk[d3n2], acc.at[didx[i4n3]], ssems[d3n2]).wait()
                    idx_load(i4n3, gn3)
                    idx_drain(i4n2, gn2)
                    pltpu.async_copy(
                        hsr.at[sidx[i4n2]], dbk[d3n2], gsems[d3n2])
                return _
            lax.fori_loop(0, ngr // 12, jbody, None)
            pltpu.make_async_copy(hsr.at[sidx[0]], dbk[0], gsems[0]).wait()
            pltpu.make_async_copy(hsr.at[sidx[1]], dbk[1], gsems[1]).wait()
            pltpu.make_async_copy(dbk[2], acc.at[didx[3]], ssems[2]).wait()
            idx_drain(2, 2)
            plsc.subcore_barrier()

            pltpu.sync_copy(acc.at[pl.ds(tb, rows_pt)],
                            outr.at[pl.ds(tb, rows_pt)])
        pl.when(c == 0)(lambda: run(hs0r, out0r))
        pl.when(c == 1)(lambda: run(hs1r, out1r))

    return spmm(hs0, hs1, src2, dst2, zer)


def _dot(a, b):
    return jnp.dot(a, b, preferred_element_type=_F32,
                   precision=lax.Precision.HIGHEST)


_BR = 1000


def _t0_call(x, d0, d1, w0):
    n, f = x.shape
    nblk = n // _BR

    def body(x_ref, d0_ref, d1_ref, w_ref, hs0_ref, hs1_ref):
        dv = d0_ref[...][:, :1] + d1_ref[...][:, :1]
        dinv = lax.rsqrt(dv)
        hs = _dot(x_ref[...], w_ref[...]) * dinv
        hs0_ref[...] = hs[:, :128]
        hs1_ref[...] = hs[:, 128:]

    return pl.pallas_call(
        body,
        grid=(nblk,),
        in_specs=[
            pl.BlockSpec((_BR, f), lambda i: (i, 0)),
            pl.BlockSpec((_BR, 16), lambda i: (i, 0)),
            pl.BlockSpec((_BR, 16), lambda i: (i, 0)),
            pl.BlockSpec((f, f), lambda i: (0, 0)),
        ],
        out_specs=[pl.BlockSpec((_BR, 128), lambda i: (i, 0))] * 2,
        out_shape=[jax.ShapeDtypeStruct((n, 128), _F32)] * 2,
    )(x, d0, d1, w0)


def _segmax_update(g_ref, h, btb, nseg, first):
    @pl.when(first)
    def _():
        g_ref[...] = jnp.full(g_ref.shape, -jnp.inf, _F32)
    smin = jnp.min(btb)
    smax = jnp.max(btb)
    for sg in range(nseg):
        def upd(sg=sg):
            mx = jnp.max(jnp.where(btb == sg, h, -jnp.inf), axis=0)
            g_ref[sg, :] = jnp.maximum(g_ref[sg, :], mx)
        pl.when((sg >= smin) & (sg <= smax))(upd)


def _tmid_call(a0, a1, d0, d1, b, w, bt16, nseg):
    n = a0.shape[0]
    f = w.shape[0]
    nblk = n // _BR

    def body(a0_ref, a1_ref, d0_ref, d1_ref, b_ref, w_ref, bt_ref,
             g_ref, hs0_ref, hs1_ref):
        dv = d0_ref[...][:, :1] + d1_ref[...][:, :1]
        dinv = lax.rsqrt(dv)
        s2 = dinv / dv
        a = jnp.concatenate([a0_ref[...], a1_ref[...]], axis=1)
        h = jnp.maximum(a * s2 + b_ref[...], 0.0)
        _segmax_update(g_ref, h, bt_ref[...][:, :1], nseg,
                       pl.program_id(0) == 0)
        hs = _dot(h, w_ref[...]) * dinv
        hs0_ref[...] = hs[:, :128]
        hs1_ref[...] = hs[:, 128:]

    return pl.pallas_call(
        body,
        grid=(nblk,),
        in_specs=[
            pl.BlockSpec((_BR, 128), lambda i: (i, 0)),
            pl.BlockSpec((_BR, 128), lambda i: (i, 0)),
            pl.BlockSpec((_BR, 16), lambda i: (i, 0)),
            pl.BlockSpec((_BR, 16), lambda i: (i, 0)),
            pl.BlockSpec((1, f), lambda i: (0, 0)),
            pl.BlockSpec((f, f), lambda i: (0, 0)),
            pl.BlockSpec((_BR, 16), lambda i: (i, 0)),
        ],
        out_specs=[
            pl.BlockSpec((nseg, f), lambda i: (0, 0)),
            pl.BlockSpec((_BR, 128), lambda i: (i, 0)),
            pl.BlockSpec((_BR, 128), lambda i: (i, 0)),
        ],
        out_shape=[
            jax.ShapeDtypeStruct((nseg, f), _F32),
            jax.ShapeDtypeStruct((n, 128), _F32),
            jax.ShapeDtypeStruct((n, 128), _F32),
        ],
    )(a0, a1, d0, d1, b, w, bt16)


def _fpool_call(a0, a1, d0, d1, b4, bt16, nseg):
    n = a0.shape[0]
    f = b4.shape[1]
    nblk = n // _BR

    def body(a0_ref, a1_ref, d0_ref, d1_ref, b_ref, bt_ref, g_ref):
        dv = d0_ref[...][:, :1] + d1_ref[...][:, :1]
        dinv = lax.rsqrt(dv)
        s2 = dinv / dv
        a = jnp.concatenate([a0_ref[...], a1_ref[...]], axis=1)
        h4 = jnp.maximum(a * s2 + b_ref[...], 0.0)
        _segmax_update(g_ref, h4, bt_ref[...][:, :1], nseg,
                       pl.program_id(0) == 0)

    return pl.pallas_call(
        body,
        grid=(nblk,),
        in_specs=[
            pl.BlockSpec((_BR, 128), lambda i: (i, 0)),
            pl.BlockSpec((_BR, 128), lambda i: (i, 0)),
            pl.BlockSpec((_BR, 16), lambda i: (i, 0)),
            pl.BlockSpec((_BR, 16), lambda i: (i, 0)),
            pl.BlockSpec((1, f), lambda i: (0, 0)),
            pl.BlockSpec((_BR, 16), lambda i: (i, 0)),
        ],
        out_specs=pl.BlockSpec((nseg, f), lambda i: (0, 0)),
        out_shape=jax.ShapeDtypeStruct((nseg, f), _F32),
    )(a0, a1, d0, d1, b4, bt16)


def _layer_norm(x, gamma, beta, eps=1e-5):
    mu = jnp.mean(x, axis=-1, keepdims=True)
    var = jnp.mean((x - mu) ** 2, axis=-1, keepdims=True)
    return (x - mu) * lax.rsqrt(var + eps) * gamma + beta


def _fhead_call(g, pi, wg, bg, gg, betg, wp, bp, gp, betp, wf, bf, gf, betf):
    nb = g.shape[0]
    nout = wf.shape[1]

    def body(g_ref, pi_ref, wg_ref, bg_ref, gg_ref, betg_ref, wp_ref, bp_ref,
             gp_ref, betp_ref, wf_ref, bf_ref, gf_ref, betf_ref, out_ref):
        ge = _dot(g_ref[...], wg_ref[...]) + bg_ref[...]
        ge = jnp.maximum(_layer_norm(ge, gg_ref[...], betg_ref[...]), 0.0)
        pe = _dot(pi_ref[...], wp_ref[...]) + bp_ref[...]
        pe = jnp.maximum(_layer_norm(pe, gp_ref[...], betp_ref[...]), 0.0)
        z = jnp.concatenate([ge, pe], axis=1)
        out = _dot(z, wf_ref[...]) + bf_ref[...]
        out_ref[...] = _layer_norm(out, gf_ref[...], betf_ref[...])

    whole = lambda a: pl.BlockSpec(a.shape, lambda: tuple(0 for _ in a.shape))
    args = (g, pi, wg, bg, gg, betg, wp, bp, gp, betp, wf, bf, gf, betf)
    return pl.pallas_call(
        body,
        in_specs=[whole(a) for a in args],
        out_specs=pl.BlockSpec((nb, nout), lambda: (0, 0)),
        out_shape=jax.ShapeDtypeStruct((nb, nout), _F32),
    )(*args)


def kernel(x, edge_index, batch, pi, W, bconv, Wg, bg, gg, betg,
           Wp, bp, gp, betp, Wf, bf, gf, betf):
    n, f = x.shape
    nlayers = W.shape[0]
    nseg = pi.shape[0]
    idt = edge_index.dtype
    ec = edge_index.shape[1] + n

    ea = 12 * _CKE * 2 * _ECK // math.gcd(12 * _CKE, 2 * _ECK)
    ept = _ceil_to(_ceil_to(ec, _NS) // _NS, ea)
    ept32 = ept // 2
    epad = _NS * ept
    rows_pt = _ceil_to(_ceil_to(n + 1, _NS) // _NS, _ECK)
    rpad = _NS * rows_pt
    zer = jnp.zeros((rows_pt, 128), _F32)
    loops = jnp.arange(n, dtype=idt)
    padn = epad - ec
    srcp = jnp.concatenate([edge_index[0], loops,
                            jnp.zeros((padn,), dtype=idt)])
    dstp = jnp.concatenate([edge_index[1], loops,
                            jnp.full((padn,), n, dtype=idt)])

    deg0, deg1 = _deg_call(dstp, rpad, ept32)
    d0, d1 = deg0[:n], deg1[:n]

    bt16 = jnp.broadcast_to(batch[:, None], (n, 16)).astype(jnp.int32)
    hs0, hs1 = _t0_call(x, d0, d1, W[0])
    gparts = []
    a0 = a1 = None
    for l in range(nlayers):
        o0, o1 = _spmm_call(hs0, hs1, srcp, dstp, zer, rpad, ept)
        a0, a1 = o0[:n], o1[:n]
        if l < nlayers - 1:
            gpart, hs0, hs1 = _tmid_call(a0, a1, d0, d1,
                                         bconv[l].reshape(1, -1), W[l + 1],
                                         bt16, nseg)
            gparts.append(gpart)

    gparts.append(_fpool_call(a0, a1, d0, d1,
                              bconv[nlayers - 1].reshape(1, -1), bt16, nseg))
    g = jnp.concatenate(gparts, axis=1)

    pf = pi.shape[1]
    pfp = _ceil_to(pf, 128)
    pi_p = jnp.pad(pi, ((0, 0), (0, pfp - pf)))
    wp_p = jnp.pad(Wp, ((0, pfp - pf), (0, 0)))

    return _fhead_call(g, pi_p, Wg, bg.reshape(1, -1), gg.reshape(1, -1),
                       betg.reshape(1, -1), wp_p, bp.reshape(1, -1),
                       gp.reshape(1, -1), betp.reshape(1, -1), Wf,
                       bf.reshape(1, -1), gf.reshape(1, -1),
                       betf.reshape(1, -1))


# speedup vs baseline: 1.1790x; 1.1790x over previous
import functools
import math

import jax
import jax.numpy as jnp
from jax import lax
from jax.experimental import pallas as pl
from jax.experimental.pallas import tpu as pltpu
from jax.experimental.pallas import tpu_sc as plsc

_F32 = jnp.float32
_NC, _NS = 2, 16
_CK = 128


def _ceil_to(a, m):
    return (a + m - 1) // m * m


def _mesh():
    return plsc.VectorSubcoreMesh(
        core_axis_name="c", subcore_axis_name="s",
        num_cores=_NC, num_subcores=_NS)


_ECK = 128
_K = 1
_CKE = 112
_ZC = 64


def _deg_call(dstp, rpad, ept32):
    ngr = ept32 // (_K * _ECK)
    rows_pt = rpad // _NS
    nch_r = rows_pt // _ECK

    @functools.partial(
        pl.kernel,
        out_type=(jax.ShapeDtypeStruct((rpad, 16), _F32),
                  jax.ShapeDtypeStruct((rpad, 16), _F32)),
        mesh=_mesh(),
        scratch_types=(
            [pltpu.VMEM((_ECK,), jnp.int32)] * _K
            + [pltpu.VMEM((_ECK, 16), _F32)] * 2
            + [pltpu.VMEM_SHARED((rpad, 16), _F32),
               pltpu.SemaphoreType.DMA]
        ),
    )
    def degk(dstr, out0r, out1r, db0, zbuf, obuf, acc, ssem):
        dbufs = (db0,)
        c = lax.axis_index("c")
        s = lax.axis_index("s")
        tb = s * rows_pt
        eb = (c * _NS + s) * ept32

        def fill(i, _):
            zbuf[i, pl.ds(0, 16)] = jnp.zeros((16,), _F32)
            obuf[i, pl.ds(0, 16)] = jnp.ones((16,), _F32)
            return _
        lax.fori_loop(0, _ECK, fill, None)

        def zcp(k, _):
            pltpu.sync_copy(zbuf, acc.at[pl.ds(tb + k * _ECK, _ECK)])
            return _
        lax.fori_loop(0, nch_r, zcp, None)
        plsc.subcore_barrier()

        def step(g, _):
            for k in range(_K):
                pltpu.sync_copy(
                    dstr.at[pl.ds(eb + (g * _K + k) * _ECK, _ECK)], dbufs[k])
            descs = [pltpu.async_copy(obuf, acc.at[dbufs[k]], ssem, add=True)
                     for k in range(_K)]
            for d in descs:
                d.wait()
            return _
        lax.fori_loop(0, ngr, step, None)
        plsc.subcore_barrier()

        def wout(outr):
            def w1(k, _):
                r0 = tb + k * _ECK
                pltpu.sync_copy(acc.at[pl.ds(r0, _ECK)], zbuf)
                pltpu.sync_copy(zbuf, outr.at[pl.ds(r0, _ECK)])
                return _
            lax.fori_loop(0, nch_r, w1, None)
        pl.when(c == 0)(lambda: wout(out0r))
        pl.when(c == 1)(lambda: wout(out1r))

    return degk(dstp)


def _spmm_call(hs0, hs1, src2, dst2, rpad, ept):
    ngr = ept // _CKE
    rows_pt = rpad // _NS
    nzw = rows_pt // _ZC

    @functools.partial(
        pl.kernel,
        out_type=(jax.ShapeDtypeStruct((rpad, 128), _F32),
                  jax.ShapeDtypeStruct((rpad, 128), _F32)),
        mesh=_mesh(),
        scratch_types=(
            [pltpu.VMEM((_CKE, 128), _F32)] * 3
            + [pltpu.VMEM((_CKE,), jnp.int32)] * 9
            + [pltpu.VMEM_SHARED((rpad, 128), _F32)]
            + [pltpu.SemaphoreType.DMA] * 10
        ),
    )
    def spmm(hs0r, hs1r, srcr, dstr, out0r, out1r,
             b0, b1, b2, sb0, sb1, sb2, sb3, db0, db1, db2, db3, tbuf,
             acc, gsm0, gsm1, gsm2, ssm0, ssm1, ssm2,
             ism0, ism1, ism2, ism3):
        dbk = (b0, b1, b2)
        gsems = (gsm0, gsm1, gsm2)
        ssems = (ssm0, ssm1, ssm2)
        sidx = (sb0, sb1, sb2, sb3)
        didx = (db0, db1, db2, db3)
        isems = (ism0, ism1, ism2, ism3)
        c = lax.axis_index("c")
        s = lax.axis_index("s")
        tb = s * rows_pt
        eb = s * ept
        trash = rpad - 1

        zb = b0

        def zrow(i, _):
            zb[i // 8, pl.ds((i % 8) * 16, 16)] = jnp.zeros((16,), _F32)
            return _
        lax.fori_loop(0, _ZC * 8, zrow, None)

        def tfill(i, _):
            tbuf[pl.ds(i * 16, 16)] = jnp.full((16,), trash, jnp.int32)
            return _
        lax.fori_loop(0, _CKE // 16, tfill, None)

        def zcp(k, _):
            pltpu.sync_copy(zb.at[pl.ds(0, _ZC)],
                            acc.at[pl.ds(tb + k * _ZC, _ZC)])
            return _
        lax.fori_loop(0, nzw, zcp, None)
        plsc.subcore_barrier()

        def idx_load(i4, g):
            pltpu.async_copy(srcr.at[pl.ds(eb + g * _CKE, _CKE)],
                             sidx[i4], isems[i4])
            pltpu.async_copy(dstr.at[pl.ds(eb + g * _CKE, _CKE)],
                             didx[i4], isems[i4])

        def idx_drain(i4, g):
            pltpu.make_async_copy(srcr.at[pl.ds(eb + g * _CKE, _CKE)],
                                  sidx[i4], isems[i4]).wait()
            pltpu.make_async_copy(dstr.at[pl.ds(eb + g * _CKE, _CKE)],
                                  didx[i4], isems[i4]).wait()

        def run(hsr, outr):
            idx_load(0, 0)
            idx_load(1, 1)
            idx_load(2, 2)
            idx_drain(0, 0)
            pltpu.async_copy(hsr.at[sidx[0]], dbk[0], gsems[0])
            idx_drain(1, 1)
            pltpu.async_copy(hsr.at[sidx[1]], dbk[1], gsems[1])
            pltpu.async_copy(dbk[2], acc.at[tbuf], ssems[2], add=True)

            def jbody(j, _):
                for p in range(12):
                    g = j * 12 + p
                    d3, i4 = p % 3, p % 4
                    d3n2, i4n2, i4n3 = (p + 2) % 3, (p + 2) % 4, (p + 3) % 4
                    gn2 = lax.rem(g + 2, ngr)
                    gn3 = lax.rem(g + 3, ngr)
                    pltpu.make_async_copy(
                        hsr.at[sidx[i4]], dbk[d3], gsems[d3]).wait()
                    pltpu.async_copy(
                        dbk[d3], acc.at[didx[i4]], ssems[d3], add=True)
                    pltpu.make_async_copy(
                        dbk[d3n2], acc.at[didx[i4n3]], ssems[d3n2]).wait()
                    idx_load(i4n3, gn3)
                    idx_drain(i4n2, gn2)
                    pltpu.async_copy(
                        hsr.at[sidx[i4n2]], dbk[d3n2], gsems[d3n2])
                return _
            lax.fori_loop(0, ngr // 12, jbody, None)
            pltpu.make_async_copy(hsr.at[sidx[0]], dbk[0], gsems[0]).wait()
            pltpu.make_async_copy(hsr.at[sidx[1]], dbk[1], gsems[1]).wait()
            pltpu.make_async_copy(dbk[2], acc.at[didx[3]], ssems[2]).wait()
            idx_drain(2, 2)
            plsc.subcore_barrier()

            def wout(k, _):
                r0 = tb + k * _ZC
                pltpu.sync_copy(acc.at[pl.ds(r0, _ZC)], b1.at[pl.ds(0, _ZC)])
                pltpu.sync_copy(b1.at[pl.ds(0, _ZC)], outr.at[pl.ds(r0, _ZC)])
                return _
            lax.fori_loop(0, nzw, wout, None)
        pl.when(c == 0)(lambda: run(hs0r, out0r))
        pl.when(c == 1)(lambda: run(hs1r, out1r))

    return spmm(hs0, hs1, src2, dst2)


def _dot(a, b):
    return jnp.dot(a, b, preferred_element_type=_F32,
                   precision=lax.Precision.HIGHEST)


_BR = 1000


def _t0_call(x, d0, d1, w0):
    n, f = x.shape
    nblk = n // _BR

    def body(x_ref, d0_ref, d1_ref, w_ref, hs0_ref, hs1_ref):
        dv = d0_ref[...][:, :1] + d1_ref[...][:, :1]
        dinv = lax.rsqrt(dv)
        hs = _dot(x_ref[...], w_ref[...]) * dinv
        hs0_ref[...] = hs[:, :128]
        hs1_ref[...] = hs[:, 128:]

    return pl.pallas_call(
        body,
        grid=(nblk,),
        in_specs=[
            pl.BlockSpec((_BR, f), lambda i: (i, 0)),
            pl.BlockSpec((_BR, 16), lambda i: (i, 0)),
            pl.BlockSpec((_BR, 16), lambda i: (i, 0)),
            pl.BlockSpec((f, f), lambda i: (0, 0)),
        ],
        out_specs=[pl.BlockSpec((_BR, 128), lambda i: (i, 0))] * 2,
        out_shape=[jax.ShapeDtypeStruct((n, 128), _F32)] * 2,
    )(x, d0, d1, w0)


def _segmax_update(g_ref, h, btb, nseg, first):
    @pl.when(first)
    def _():
        g_ref[...] = jnp.full(g_ref.shape, -jnp.inf, _F32)
    smin = jnp.min(btb)
    smax = jnp.max(btb)
    for sg in range(nseg):
        def upd(sg=sg):
            mx = jnp.max(jnp.where(btb == sg, h, -jnp.inf), axis=0)
            g_ref[sg, :] = jnp.maximum(g_ref[sg, :], mx)
        pl.when((sg >= smin) & (sg <= smax))(upd)


def _tmid_call(a0, a1, d0, d1, b, w, bt16, nseg):
    n = a0.shape[0]
    f = w.shape[0]
    nblk = n // _BR

    def body(a0_ref, a1_ref, d0_ref, d1_ref, b_ref, w_ref, bt_ref,
             g_ref, hs0_ref, hs1_ref):
        dv = d0_ref[...][:, :1] + d1_ref[...][:, :1]
        dinv = lax.rsqrt(dv)
        s2 = dinv / dv
        a = jnp.concatenate([a0_ref[...], a1_ref[...]], axis=1)
        h = jnp.maximum(a * s2 + b_ref[...], 0.0)
        _segmax_update(g_ref, h, bt_ref[...][:, :1], nseg,
                       pl.program_id(0) == 0)
        hs = _dot(h, w_ref[...]) * dinv
        hs0_ref[...] = hs[:, :128]
        hs1_ref[...] = hs[:, 128:]

    return pl.pallas_call(
        body,
        grid=(nblk,),
        in_specs=[
            pl.BlockSpec((_BR, 128), lambda i: (i, 0)),
            pl.BlockSpec((_BR, 128), lambda i: (i, 0)),
            pl.BlockSpec((_BR, 16), lambda i: (i, 0)),
            pl.BlockSpec((_BR, 16), lambda i: (i, 0)),
            pl.BlockSpec((1, f), lambda i: (0, 0)),
            pl.BlockSpec((f, f), lambda i: (0, 0)),
            pl.BlockSpec((_BR, 16), lambda i: (i, 0)),
        ],
        out_specs=[
            pl.BlockSpec((nseg, f), lambda i: (0, 0)),
            pl.BlockSpec((_BR, 128), lambda i: (i, 0)),
            pl.BlockSpec((_BR, 128), lambda i: (i, 0)),
        ],
        out_shape=[
            jax.ShapeDtypeStruct((nseg, f), _F32),
            jax.ShapeDtypeStruct((n, 128), _F32),
            jax.ShapeDtypeStruct((n, 128), _F32),
        ],
    )(a0, a1, d0, d1, b, w, bt16)


def _fpool_call(a0, a1, d0, d1, b4, bt16, nseg):
    n = a0.shape[0]
    f = b4.shape[1]
    nblk = n // _BR

    def body(a0_ref, a1_ref, d0_ref, d1_ref, b_ref, bt_ref, g_ref):
        dv = d0_ref[...][:, :1] + d1_ref[...][:, :1]
        dinv = lax.rsqrt(dv)
        s2 = dinv / dv
        a = jnp.concatenate([a0_ref[...], a1_ref[...]], axis=1)
        h4 = jnp.maximum(a * s2 + b_ref[...], 0.0)
        _segmax_update(g_ref, h4, bt_ref[...][:, :1], nseg,
                       pl.program_id(0) == 0)

    return pl.pallas_call(
        body,
        grid=(nblk,),
        in_specs=[
            pl.BlockSpec((_BR, 128), lambda i: (i, 0)),
            pl.BlockSpec((_BR, 128), lambda i: (i, 0)),
            pl.BlockSpec((_BR, 16), lambda i: (i, 0)),
            pl.BlockSpec((_BR, 16), lambda i: (i, 0)),
            pl.BlockSpec((1, f), lambda i: (0, 0)),
            pl.BlockSpec((_BR, 16), lambda i: (i, 0)),
        ],
        out_specs=pl.BlockSpec((nseg, f), lambda i: (0, 0)),
        out_shape=jax.ShapeDtypeStruct((nseg, f), _F32),
    )(a0, a1, d0, d1, b4, bt16)


def _layer_norm(x, gamma, beta, eps=1e-5):
    mu = jnp.mean(x, axis=-1, keepdims=True)
    var = jnp.mean((x - mu) ** 2, axis=-1, keepdims=True)
    return (x - mu) * lax.rsqrt(var + eps) * gamma + beta


def _fhead_call(g, pi, wg, bg, gg, betg, wp, bp, gp, betp, wf, bf, gf, betf):
    nb = g.shape[0]
    nout = wf.shape[1]

    def body(g_ref, pi_ref, wg_ref, bg_ref, gg_ref, betg_ref, wp_ref, bp_ref,
             gp_ref, betp_ref, wf_ref, bf_ref, gf_ref, betf_ref, out_ref):
        ge = _dot(g_ref[...], wg_ref[...]) + bg_ref[...]
        ge = jnp.maximum(_layer_norm(ge, gg_ref[...], betg_ref[...]), 0.0)
        pe = _dot(pi_ref[...], wp_ref[...]) + bp_ref[...]
        pe = jnp.maximum(_layer_norm(pe, gp_ref[...], betp_ref[...]), 0.0)
        z = jnp.concatenate([ge, pe], axis=1)
        out = _dot(z, wf_ref[...]) + bf_ref[...]
        out_ref[...] = _layer_norm(out, gf_ref[...], betf_ref[...])

    whole = lambda a: pl.BlockSpec(a.shape, lambda: tuple(0 for _ in a.shape))
    args = (g, pi, wg, bg, gg, betg, wp, bp, gp, betp, wf, bf, gf, betf)
    return pl.pallas_call(
        body,
        in_specs=[whole(a) for a in args],
        out_specs=pl.BlockSpec((nb, nout), lambda: (0, 0)),
        out_shape=jax.ShapeDtypeStruct((nb, nout), _F32),
    )(*args)


def kernel(x, edge_index, batch, pi, W, bconv, Wg, bg, gg, betg,
           Wp, bp, gp, betp, Wf, bf, gf, betf):
    n, f = x.shape
    nlayers = W.shape[0]
    nseg = pi.shape[0]
    idt = edge_index.dtype
    ec = edge_index.shape[1] + n

    ea = 12 * _CKE * 2 * _ECK // math.gcd(12 * _CKE, 2 * _ECK)
    ept = _ceil_to(_ceil_to(ec, _NS) // _NS, ea)
    ept32 = ept // 2
    epad = _NS * ept
    rows_pt = _ceil_to(_ceil_to(n + 1, _NS) // _NS, _ECK)
    rpad = _NS * rows_pt
    loops = jnp.arange(n, dtype=idt)
    padn = epad - ec
    srcp = jnp.concatenate([edge_index[0], loops,
                            jnp.zeros((padn,), dtype=idt)])
    dstp = jnp.concatenate([edge_index[1], loops,
                            jnp.full((padn,), n, dtype=idt)])

    deg0, deg1 = _deg_call(dstp, rpad, ept32)
    d0, d1 = deg0[:n], deg1[:n]

    bt16 = jnp.broadcast_to(batch[:, None], (n, 16)).astype(jnp.int32)
    hs0, hs1 = _t0_call(x, d0, d1, W[0])
    gparts = []
    a0 = a1 = None
    for l in range(nlayers):
        o0, o1 = _spmm_call(hs0, hs1, srcp, dstp, rpad, ept)
        a0, a1 = o0[:n], o1[:n]
        if l < nlayers - 1:
            gpart, hs0, hs1 = _tmid_call(a0, a1, d0, d1,
                                         bconv[l].reshape(1, -1), W[l + 1],
                                         bt16, nseg)
            gparts.append(gpart)

    gparts.append(_fpool_call(a0, a1, d0, d1,
                              bconv[nlayers - 1].reshape(1, -1), bt16, nseg))
    g = jnp.concatenate(gparts, axis=1)

    pf = pi.shape[1]
    pfp = _ceil_to(pf, 128)
    pi_p = jnp.pad(pi, ((0, 0), (0, pfp - pf)))
    wp_p = jnp.pad(Wp, ((0, pfp - pf), (0, 0)))

    return _fhead_call(g, pi_p, Wg, bg.reshape(1, -1), gg.reshape(1, -1),
                       betg.reshape(1, -1), wp_p, bp.reshape(1, -1),
                       gp.reshape(1, -1), betp.reshape(1, -1), Wf,
                       bf.reshape(1, -1), gf.reshape(1, -1),
                       betf.reshape(1, -1))
